# Initial kernel scaffold; baseline (speedup 1.0000x reference)
import jax
import jax.numpy as jnp
from jax.experimental import pallas as pl


def kernel(prev_ids, cur_ids, embed, proj_w):
    raise NotImplementedError("write your pallas kernel here")


# speedup vs baseline: 16.7445x; 16.7445x over previous
import functools

import numpy as np

import jax
import jax.numpy as jnp
from jax import lax
from jax.experimental import pallas as pl
from jax.experimental.pallas import tpu as pltpu
from jax.experimental.pallas import tpu_sc as plsc

LANES = 16
CHUNK = 128


def _sc_gather(prev, cur, embed, num_buckets, mult):
    n = prev.shape[0]
    d = embed.shape[1]
    info = plsc.get_sparse_core_info()
    nc, ns = info.num_cores, info.num_subcores
    nw = nc * ns
    n_per_w = n // nw
    n_chunks = n_per_w // CHUNK

    mesh = plsc.VectorSubcoreMesh(core_axis_name="c", subcore_axis_name="s")

    @functools.partial(
        pl.kernel,
        out_type=jax.ShapeDtypeStruct((n, d), jnp.float32),
        mesh=mesh,
        scratch_types=[
            pltpu.VMEM((n_per_w,), jnp.int32),
            pltpu.VMEM((n_per_w,), jnp.int32),
            pltpu.VMEM((n_per_w,), jnp.int32),
            pltpu.VMEM((2, CHUNK, d), jnp.float32),
            pltpu.SemaphoreType.DMA,
            pltpu.SemaphoreType.DMA,
        ],
        compiler_params=pltpu.CompilerParams(use_tc_tiling_on_sc=False),
    )
    def sc_kernel(prev_hbm, cur_hbm, embed_hbm, e_hbm,
                  prev_v, cur_v, idx_v, rows_v, gsem, ssem):
        wid = lax.axis_index("s") * jnp.int32(nc) + lax.axis_index("c")
        base = pl.multiple_of(wid * jnp.int32(n_per_w), n_per_w)
        pltpu.sync_copy(prev_hbm.at[pl.ds(base, n_per_w)], prev_v)
        pltpu.sync_copy(cur_hbm.at[pl.ds(base, n_per_w)], cur_v)

        two_nb = jnp.int32(2 * num_buckets)
        one_nb = jnp.int32(num_buckets)

        @pl.loop(np.int32(0), np.int32(n_per_w // LANES), unroll=4)
        def hash_body(i):
            off = pl.multiple_of(i * jnp.int32(LANES), LANES)
            h = prev_v[pl.ds(off, LANES)] * jnp.int32(mult) + cur_v[pl.ds(off, LANES)]
            h = h - jnp.where(h >= two_nb, two_nb, jnp.int32(0))
            h = h - jnp.where(h >= one_nb, one_nb, jnp.int32(0))
            idx_v[pl.ds(off, LANES)] = h

        def start_gather(c, slot):
            off = pl.multiple_of(c * jnp.int32(CHUNK), CHUNK)
            return pltpu.async_copy(
                embed_hbm.at[idx_v.at[pl.ds(off, CHUNK)]], rows_v.at[slot], gsem)

        start_gather(0, 0).wait()

        @pl.loop(np.int32(0), np.int32(n_chunks - 1))
        def loop_body(c):
            slot = lax.rem(c, jnp.int32(2))
            nxt = jnp.int32(1) - slot
            cp = start_gather(c + jnp.int32(1), nxt)
            off = pl.multiple_of(base + c * jnp.int32(CHUNK), CHUNK)
            pltpu.async_copy(rows_v.at[slot], e_hbm.at[pl.ds(off, CHUNK)], ssem).wait()
            cp.wait()
        last = n_chunks - 1
        off = pl.multiple_of(base + jnp.int32(last * CHUNK), CHUNK)
        pltpu.async_copy(rows_v.at[last % 2],
                         e_hbm.at[pl.ds(off, CHUNK)], ssem).wait()

    return sc_kernel(prev, cur, embed)


def _tc_project(e, proj_w, block_m=4096):
    n, d = e.shape
    m = proj_w.shape[0]

    def mm_body(e_ref, w_ref, o_ref):
        o_ref[...] = lax.dot_general(
            e_ref[...], w_ref[...], (((1,), (1,)), ((), ())),
            preferred_element_type=jnp.float32)

    return pl.pallas_call(
        mm_body,
        grid=(n // block_m,),
        in_specs=[
            pl.BlockSpec((block_m, d), lambda i: (i, 0)),
            pl.BlockSpec((m, d), lambda i: (0, 0)),
        ],
        out_specs=pl.BlockSpec((block_m, m), lambda i: (i, 0)),
        out_shape=jax.ShapeDtypeStruct((n, m), jnp.float32),
    )(e, proj_w)


def kernel(prev_ids, cur_ids, embed, proj_w):
    with jax._src.config.enable_x64(False):
        b, l = prev_ids.shape
        num_buckets = embed.shape[0]
        mult = 1000003 % num_buckets
        prev = prev_ids.astype(jnp.int32).reshape(-1)
        cur = cur_ids.astype(jnp.int32).reshape(-1)
        e = _sc_gather(prev, cur, embed, num_buckets, mult)
        out = _tc_project(e, proj_w)
        return out.reshape(b, l, proj_w.shape[0])


# speedup vs baseline: 19.3776x; 1.1572x over previous
import functools

import numpy as np

import jax
import jax.numpy as jnp
from jax import lax
from jax.experimental import pallas as pl
from jax.experimental.pallas import tpu as pltpu
from jax.experimental.pallas import tpu_sc as plsc

LANES = 16
CHUNK = 128


def _sc_gather(prev, cur, embed, num_buckets, mult):
    n = prev.shape[0]
    d = embed.shape[1]
    info = plsc.get_sparse_core_info()
    nc, ns = info.num_cores, info.num_subcores
    nw = nc * ns
    n_per_w = n // nw
    n_chunks = n_per_w // CHUNK

    mesh = plsc.VectorSubcoreMesh(core_axis_name="c", subcore_axis_name="s")

    @functools.partial(
        pl.kernel,
        out_type=jax.ShapeDtypeStruct((n, d), jnp.float32),
        mesh=mesh,
        scratch_types=[
            pltpu.VMEM((n_per_w,), jnp.int32),
            pltpu.VMEM((n_per_w,), jnp.int32),
            pltpu.VMEM((n_per_w,), jnp.int32),
            pltpu.VMEM((2, CHUNK, d), jnp.float32),
            pltpu.SemaphoreType.DMA,
            pltpu.SemaphoreType.DMA,
        ],
        compiler_params=pltpu.CompilerParams(use_tc_tiling_on_sc=False),
    )
    def sc_kernel(prev_hbm, cur_hbm, embed_hbm, e_hbm,
                  prev_v, cur_v, idx_v, rows_v, gsem, ssem):
        wid = lax.axis_index("s") * jnp.int32(nc) + lax.axis_index("c")
        base = pl.multiple_of(wid * jnp.int32(n_per_w), n_per_w)
        pltpu.sync_copy(prev_hbm.at[pl.ds(base, n_per_w)], prev_v)
        pltpu.sync_copy(cur_hbm.at[pl.ds(base, n_per_w)], cur_v)

        two_nb = jnp.int32(2 * num_buckets)
        one_nb = jnp.int32(num_buckets)

        @pl.loop(np.int32(0), np.int32(n_per_w // LANES), unroll=4)
        def hash_body(i):
            off = pl.multiple_of(i * jnp.int32(LANES), LANES)
            h = prev_v[pl.ds(off, LANES)] * jnp.int32(mult) + cur_v[pl.ds(off, LANES)]
            h = h - jnp.where(h >= two_nb, two_nb, jnp.int32(0))
            h = h - jnp.where(h >= one_nb, one_nb, jnp.int32(0))
            idx_v[pl.ds(off, LANES)] = h

        def start_gather(c, slot):
            off = pl.multiple_of(c * jnp.int32(CHUNK), CHUNK)
            return pltpu.async_copy(
                embed_hbm.at[idx_v.at[pl.ds(off, CHUNK)]], rows_v.at[slot], gsem)

        start_gather(0, 0).wait()

        @pl.loop(np.int32(0), np.int32(n_chunks - 1))
        def loop_body(c):
            slot = lax.rem(c, jnp.int32(2))
            nxt = jnp.int32(1) - slot
            cp = start_gather(c + jnp.int32(1), nxt)
            off = pl.multiple_of(base + c * jnp.int32(CHUNK), CHUNK)
            pltpu.async_copy(rows_v.at[slot], e_hbm.at[pl.ds(off, CHUNK)], ssem).wait()
            cp.wait()
        last = n_chunks - 1
        off = pl.multiple_of(base + jnp.int32(last * CHUNK), CHUNK)
        pltpu.async_copy(rows_v.at[last % 2],
                         e_hbm.at[pl.ds(off, CHUNK)], ssem).wait()

    return sc_kernel(prev, cur, embed)


def _tc_project_packed(p, proj_w, block_m=2048):
    n4, dd = p.shape
    m, d = proj_w.shape
    packs = dd // d

    def mm_body(p_ref, w_ref, o_ref):
        bands = [
            lax.dot_general(
                p_ref[:, s * d:(s + 1) * d], w_ref[...],
                (((1,), (1,)), ((), ())),
                preferred_element_type=jnp.float32)
            for s in range(packs)
        ]
        o_ref[...] = jnp.stack(bands, axis=1).reshape(packs * block_m, m)

    return pl.pallas_call(
        mm_body,
        grid=(n4 // block_m,),
        in_specs=[
            pl.BlockSpec((block_m, dd), lambda i: (i, 0)),
            pl.BlockSpec((m, d), lambda i: (0, 0)),
        ],
        out_specs=pl.BlockSpec((packs * block_m, m), lambda i: (i, 0)),
        out_shape=jax.ShapeDtypeStruct((packs * n4, m), jnp.float32),
    )(p, proj_w)


def kernel(prev_ids, cur_ids, embed, proj_w):
    with jax._src.config.enable_x64(False):
        b, l = prev_ids.shape
        num_buckets = embed.shape[0]
        mult = 1000003 % num_buckets
        prev = prev_ids.astype(jnp.int32).reshape(-1)
        cur = cur_ids.astype(jnp.int32).reshape(-1)
        e = _sc_gather(prev, cur, embed, num_buckets, mult)
        d = embed.shape[1]
        packs = 128 // d
        out = _tc_project_packed(e.reshape(-1, packs * d), proj_w)
        return out.reshape(b, l, proj_w.shape[0])


# speedup vs baseline: 24.0420x; 1.2407x over previous
import functools

import numpy as np

import jax
import jax.numpy as jnp
from jax import lax
from jax.experimental import pallas as pl
from jax.experimental.pallas import tpu as pltpu
from jax.experimental.pallas import tpu_sc as plsc

LANES = 16
CHUNK = 128


def _tc_build_p(embed_t, proj_w, block_k=2048):
    d, nb = embed_t.shape
    m = proj_w.shape[0]

    def body(et_ref, w_ref, p_ref):
        pt = lax.dot_general(
            w_ref[...], et_ref[...], (((1,), (0,)), ((), ())),
            preferred_element_type=jnp.float32)
        p_ref[...] = pt.T

    grid = (nb + block_k - 1) // block_k
    return pl.pallas_call(
        body,
        grid=(grid,),
        in_specs=[
            pl.BlockSpec((d, block_k), lambda i: (0, i)),
            pl.BlockSpec((m, d), lambda i: (0, 0)),
        ],
        out_specs=pl.BlockSpec((block_k, m), lambda i: (i, 0)),
        out_shape=jax.ShapeDtypeStruct((nb, m), jnp.float32),
    )(embed_t, proj_w)


def _sc_gather(prev, cur, table, num_buckets, mult):
    n = prev.shape[0]
    m = table.shape[1]
    info = plsc.get_sparse_core_info()
    nc, ns = info.num_cores, info.num_subcores
    nw = nc * ns
    n_per_w = n // nw
    n_chunks = n_per_w // CHUNK

    mesh = plsc.VectorSubcoreMesh(core_axis_name="c", subcore_axis_name="s")

    @functools.partial(
        pl.kernel,
        out_type=jax.ShapeDtypeStruct((n, m), jnp.float32),
        mesh=mesh,
        scratch_types=[
            pltpu.VMEM((n_per_w,), jnp.int32),
            pltpu.VMEM((n_per_w,), jnp.int32),
            pltpu.VMEM((n_per_w,), jnp.int32),
            pltpu.VMEM((2, CHUNK, m), jnp.float32),
            pltpu.SemaphoreType.DMA,
            pltpu.SemaphoreType.DMA,
        ],
        compiler_params=pltpu.CompilerParams(use_tc_tiling_on_sc=False),
    )
    def sc_kernel(prev_hbm, cur_hbm, table_hbm, out_hbm,
                  prev_v, cur_v, idx_v, rows_v, gsem, ssem):
        wid = lax.axis_index("s") * jnp.int32(nc) + lax.axis_index("c")
        base = pl.multiple_of(wid * jnp.int32(n_per_w), n_per_w)
        pltpu.sync_copy(prev_hbm.at[pl.ds(base, n_per_w)], prev_v)
        pltpu.sync_copy(cur_hbm.at[pl.ds(base, n_per_w)], cur_v)

        two_nb = jnp.int32(2 * num_buckets)
        one_nb = jnp.int32(num_buckets)

        @pl.loop(np.int32(0), np.int32(n_per_w // LANES), unroll=4)
        def hash_body(i):
            off = pl.multiple_of(i * jnp.int32(LANES), LANES)
            h = prev_v[pl.ds(off, LANES)] * jnp.int32(mult) + cur_v[pl.ds(off, LANES)]
            h = h - jnp.where(h >= two_nb, two_nb, jnp.int32(0))
            h = h - jnp.where(h >= one_nb, one_nb, jnp.int32(0))
            idx_v[pl.ds(off, LANES)] = h

        def start_gather(c, slot):
            off = pl.multiple_of(c * jnp.int32(CHUNK), CHUNK)
            return pltpu.async_copy(
                table_hbm.at[idx_v.at[pl.ds(off, CHUNK)]], rows_v.at[slot], gsem)

        start_gather(0, 0).wait()

        @pl.loop(np.int32(0), np.int32(n_chunks - 1))
        def loop_body(c):
            slot = lax.rem(c, jnp.int32(2))
            nxt = jnp.int32(1) - slot
            cp = start_gather(c + jnp.int32(1), nxt)
            off = pl.multiple_of(base + c * jnp.int32(CHUNK), CHUNK)
            pltpu.async_copy(rows_v.at[slot], out_hbm.at[pl.ds(off, CHUNK)], ssem).wait()
            cp.wait()
        last = n_chunks - 1
        off = pl.multiple_of(base + jnp.int32(last * CHUNK), CHUNK)
        pltpu.async_copy(rows_v.at[last % 2],
                         out_hbm.at[pl.ds(off, CHUNK)], ssem).wait()

    return sc_kernel(prev, cur, table)


def kernel(prev_ids, cur_ids, embed, proj_w):
    with jax._src.config.enable_x64(False):
        b, l = prev_ids.shape
        num_buckets = embed.shape[0]
        mult = 1000003 % num_buckets
        prev = prev_ids.astype(jnp.int32).reshape(-1)
        cur = cur_ids.astype(jnp.int32).reshape(-1)
        p = _tc_build_p(embed.T, proj_w)
        out = _sc_gather(prev, cur, p, num_buckets, mult)
        return out.reshape(b, l, proj_w.shape[0])


# speedup vs baseline: 31.8304x; 1.3239x over previous
import functools

import numpy as np

import jax
import jax.numpy as jnp
from jax import lax
from jax.experimental import pallas as pl
from jax.experimental.pallas import tpu as pltpu
from jax.experimental.pallas import tpu_sc as plsc

LANES = 16
CHUNK = 128


def _tc_build_p(embed_t, proj_w, block_k=8192):
    d, nb = embed_t.shape
    m = proj_w.shape[0]

    def body(et_ref, w_ref, p_ref):
        p_ref[...] = lax.dot_general(
            et_ref[...], w_ref[...], (((0,), (1,)), ((), ())),
            preferred_element_type=jnp.float32)

    grid = (nb + block_k - 1) // block_k
    return pl.pallas_call(
        body,
        grid=(grid,),
        in_specs=[
            pl.BlockSpec((d, block_k), lambda i: (0, i)),
            pl.BlockSpec((m, d), lambda i: (0, 0)),
        ],
        out_specs=pl.BlockSpec((block_k, m), lambda i: (i, 0)),
        out_shape=jax.ShapeDtypeStruct((nb, m), jnp.float32),
    )(embed_t, proj_w)


def _sc_gather(prev, cur, table, num_buckets, mult):
    n = prev.shape[0]
    m = table.shape[1]
    info = plsc.get_sparse_core_info()
    nc, ns = info.num_cores, info.num_subcores
    nw = nc * ns
    n_per_w = n // nw
    n_chunks = n_per_w // CHUNK

    mesh = plsc.VectorSubcoreMesh(core_axis_name="c", subcore_axis_name="s")

    @functools.partial(
        pl.kernel,
        out_type=jax.ShapeDtypeStruct((n, m), jnp.float32),
        mesh=mesh,
        scratch_types=[
            pltpu.VMEM((n_per_w,), jnp.int32),
            pltpu.VMEM((n_per_w,), jnp.int32),
            pltpu.VMEM((n_per_w,), jnp.int32),
            pltpu.VMEM((2, CHUNK, m), jnp.float32),
            pltpu.SemaphoreType.DMA,
            pltpu.SemaphoreType.DMA,
        ],
        compiler_params=pltpu.CompilerParams(use_tc_tiling_on_sc=False),
    )
    def sc_kernel(prev_hbm, cur_hbm, table_hbm, out_hbm,
                  prev_v, cur_v, idx_v, rows_v, gsem, ssem):
        wid = lax.axis_index("s") * jnp.int32(nc) + lax.axis_index("c")
        base = pl.multiple_of(wid * jnp.int32(n_per_w), n_per_w)
        pltpu.sync_copy(prev_hbm.at[pl.ds(base, n_per_w)], prev_v)
        pltpu.sync_copy(cur_hbm.at[pl.ds(base, n_per_w)], cur_v)

        two_nb = jnp.int32(2 * num_buckets)
        one_nb = jnp.int32(num_buckets)

        @pl.loop(np.int32(0), np.int32(n_per_w // LANES), unroll=4)
        def hash_body(i):
            off = pl.multiple_of(i * jnp.int32(LANES), LANES)
            h = prev_v[pl.ds(off, LANES)] * jnp.int32(mult) + cur_v[pl.ds(off, LANES)]
            h = h - jnp.where(h >= two_nb, two_nb, jnp.int32(0))
            h = h - jnp.where(h >= one_nb, one_nb, jnp.int32(0))
            idx_v[pl.ds(off, LANES)] = h

        def start_gather(c, slot):
            off = pl.multiple_of(c * jnp.int32(CHUNK), CHUNK)
            return pltpu.async_copy(
                table_hbm.at[idx_v.at[pl.ds(off, CHUNK)]], rows_v.at[slot], gsem)

        start_gather(0, 0).wait()

        @pl.loop(np.int32(0), np.int32(n_chunks - 1))
        def loop_body(c):
            slot = lax.rem(c, jnp.int32(2))
            nxt = jnp.int32(1) - slot
            cp = start_gather(c + jnp.int32(1), nxt)
            off = pl.multiple_of(base + c * jnp.int32(CHUNK), CHUNK)
            pltpu.async_copy(rows_v.at[slot], out_hbm.at[pl.ds(off, CHUNK)], ssem).wait()
            cp.wait()
        last = n_chunks - 1
        off = pl.multiple_of(base + jnp.int32(last * CHUNK), CHUNK)
        pltpu.async_copy(rows_v.at[last % 2],
                         out_hbm.at[pl.ds(off, CHUNK)], ssem).wait()

    return sc_kernel(prev, cur, table)


def kernel(prev_ids, cur_ids, embed, proj_w):
    with jax._src.config.enable_x64(False):
        b, l = prev_ids.shape
        num_buckets = embed.shape[0]
        mult = 1000003 % num_buckets
        prev = prev_ids.astype(jnp.int32).reshape(-1)
        cur = cur_ids.astype(jnp.int32).reshape(-1)
        p = _tc_build_p(embed.T, proj_w)
        out = _sc_gather(prev, cur, p, num_buckets, mult)
        return out.reshape(b, l, proj_w.shape[0])


# speedup vs baseline: 33.6836x; 1.0582x over previous
import functools

import numpy as np

import jax
import jax.numpy as jnp
from jax import lax
from jax.experimental import pallas as pl
from jax.experimental.pallas import tpu as pltpu
from jax.experimental.pallas import tpu_sc as plsc

LANES = 16
CHUNK = 128


def _tc_build_p(embed_t, proj_w, block_k=16384):
    d, nb = embed_t.shape
    m = proj_w.shape[0]

    def body(et_ref, w_ref, p_ref):
        p_ref[...] = lax.dot_general(
            et_ref[...], w_ref[...], (((0,), (1,)), ((), ())),
            preferred_element_type=jnp.float32)

    grid = (nb + block_k - 1) // block_k
    return pl.pallas_call(
        body,
        grid=(grid,),
        in_specs=[
            pl.BlockSpec((d, block_k), lambda i: (0, i)),
            pl.BlockSpec((m, d), lambda i: (0, 0)),
        ],
        out_specs=pl.BlockSpec((block_k, m), lambda i: (i, 0)),
        out_shape=jax.ShapeDtypeStruct((nb, m), jnp.float32),
    )(embed_t, proj_w)


def _sc_gather(prev, cur, table, num_buckets, mult):
    n = prev.shape[0]
    m = table.shape[1]
    info = plsc.get_sparse_core_info()
    nc, ns = info.num_cores, info.num_subcores
    nw = nc * ns
    n_per_w = n // nw
    n_chunks = n_per_w // CHUNK

    mesh = plsc.VectorSubcoreMesh(core_axis_name="c", subcore_axis_name="s")

    @functools.partial(
        pl.kernel,
        out_type=jax.ShapeDtypeStruct((n, m), jnp.float32),
        mesh=mesh,
        scratch_types=[
            pltpu.VMEM((n_per_w,), jnp.int32),
            pltpu.VMEM((n_per_w,), jnp.int32),
            pltpu.VMEM((n_per_w,), jnp.int32),
            pltpu.VMEM((2, CHUNK, m), jnp.float32),
            pltpu.SemaphoreType.DMA,
            pltpu.SemaphoreType.DMA,
        ],
        compiler_params=pltpu.CompilerParams(use_tc_tiling_on_sc=False),
    )
    def sc_kernel(prev_hbm, cur_hbm, table_hbm, out_hbm,
                  prev_v, cur_v, idx_v, rows_v, gsem, ssem):
        wid = lax.axis_index("s") * jnp.int32(nc) + lax.axis_index("c")
        base = pl.multiple_of(wid * jnp.int32(n_per_w), n_per_w)
        pltpu.sync_copy(prev_hbm.at[pl.ds(base, n_per_w)], prev_v)
        pltpu.sync_copy(cur_hbm.at[pl.ds(base, n_per_w)], cur_v)

        two_nb = jnp.int32(2 * num_buckets)
        one_nb = jnp.int32(num_buckets)

        @pl.loop(np.int32(0), np.int32(n_per_w // LANES), unroll=4)
        def hash_body(i):
            off = pl.multiple_of(i * jnp.int32(LANES), LANES)
            h = prev_v[pl.ds(off, LANES)] * jnp.int32(mult) + cur_v[pl.ds(off, LANES)]
            h = h - jnp.where(h >= two_nb, two_nb, jnp.int32(0))
            h = h - jnp.where(h >= one_nb, one_nb, jnp.int32(0))
            idx_v[pl.ds(off, LANES)] = h

        def start_gather(c, slot):
            off = pl.multiple_of(c * jnp.int32(CHUNK), CHUNK)
            return pltpu.async_copy(
                table_hbm.at[idx_v.at[pl.ds(off, CHUNK)]], rows_v.at[slot], gsem)

        start_gather(0, 0).wait()

        @pl.loop(np.int32(0), np.int32(n_chunks - 1))
        def loop_body(c):
            slot = lax.rem(c, jnp.int32(2))
            nxt = jnp.int32(1) - slot
            cp = start_gather(c + jnp.int32(1), nxt)
            off = pl.multiple_of(base + c * jnp.int32(CHUNK), CHUNK)
            pltpu.async_copy(rows_v.at[slot], out_hbm.at[pl.ds(off, CHUNK)], ssem).wait()
            cp.wait()
        last = n_chunks - 1
        off = pl.multiple_of(base + jnp.int32(last * CHUNK), CHUNK)
        pltpu.async_copy(rows_v.at[last % 2],
                         out_hbm.at[pl.ds(off, CHUNK)], ssem).wait()

    return sc_kernel(prev, cur, table)


def kernel(prev_ids, cur_ids, embed, proj_w):
    with jax._src.config.enable_x64(False):
        b, l = prev_ids.shape
        num_buckets = embed.shape[0]
        mult = 1000003 % num_buckets
        prev = prev_ids.astype(jnp.int32).reshape(-1)
        cur = cur_ids.astype(jnp.int32).reshape(-1)
        p = _tc_build_p(embed.T, proj_w)
        out = _sc_gather(prev, cur, p, num_buckets, mult)
        return out.reshape(b, l, proj_w.shape[0])


# speedup vs baseline: 34.0521x; 1.0109x over previous
import functools

import numpy as np

import jax
import jax.numpy as jnp
from jax import lax
from jax.experimental import pallas as pl
from jax.experimental.pallas import tpu as pltpu
from jax.experimental.pallas import tpu_sc as plsc

LANES = 16
CHUNK = 128


def _tc_build_p(embed_t, proj_w, block_k=32768):
    d, nb = embed_t.shape
    m = proj_w.shape[0]

    def body(et_ref, w_ref, p_ref):
        p_ref[...] = lax.dot_general(
            et_ref[...], w_ref[...], (((0,), (1,)), ((), ())),
            preferred_element_type=jnp.float32)

    grid = (nb + block_k - 1) // block_k
    return pl.pallas_call(
        body,
        grid=(grid,),
        in_specs=[
            pl.BlockSpec((d, block_k), lambda i: (0, i)),
            pl.BlockSpec((m, d), lambda i: (0, 0)),
        ],
        out_specs=pl.BlockSpec((block_k, m), lambda i: (i, 0)),
        out_shape=jax.ShapeDtypeStruct((nb, m), jnp.float32),
    )(embed_t, proj_w)


def _sc_gather(prev, cur, table, num_buckets, mult):
    n = prev.shape[0]
    m = table.shape[1]
    info = plsc.get_sparse_core_info()
    nc, ns = info.num_cores, info.num_subcores
    nw = nc * ns
    n_per_w = n // nw
    n_chunks = n_per_w // CHUNK

    mesh = plsc.VectorSubcoreMesh(core_axis_name="c", subcore_axis_name="s")

    @functools.partial(
        pl.kernel,
        out_type=jax.ShapeDtypeStruct((n, m), jnp.float32),
        mesh=mesh,
        scratch_types=[
            pltpu.VMEM((n_per_w,), jnp.int32),
            pltpu.VMEM((n_per_w,), jnp.int32),
            pltpu.VMEM((n_per_w,), jnp.int32),
            pltpu.VMEM((2, CHUNK, m), jnp.float32),
            pltpu.SemaphoreType.DMA,
            pltpu.SemaphoreType.DMA,
        ],
        compiler_params=pltpu.CompilerParams(use_tc_tiling_on_sc=False),
    )
    def sc_kernel(prev_hbm, cur_hbm, table_hbm, out_hbm,
                  prev_v, cur_v, idx_v, rows_v, gsem, ssem):
        wid = lax.axis_index("s") * jnp.int32(nc) + lax.axis_index("c")
        base = pl.multiple_of(wid * jnp.int32(n_per_w), n_per_w)
        pltpu.sync_copy(prev_hbm.at[pl.ds(base, n_per_w)], prev_v)
        pltpu.sync_copy(cur_hbm.at[pl.ds(base, n_per_w)], cur_v)

        two_nb = jnp.int32(2 * num_buckets)
        one_nb = jnp.int32(num_buckets)

        @pl.loop(np.int32(0), np.int32(n_per_w // LANES), unroll=4)
        def hash_body(i):
            off = pl.multiple_of(i * jnp.int32(LANES), LANES)
            h = prev_v[pl.ds(off, LANES)] * jnp.int32(mult) + cur_v[pl.ds(off, LANES)]
            h = h - jnp.where(h >= two_nb, two_nb, jnp.int32(0))
            h = h - jnp.where(h >= one_nb, one_nb, jnp.int32(0))
            idx_v[pl.ds(off, LANES)] = h

        def start_gather(c, slot):
            off = pl.multiple_of(c * jnp.int32(CHUNK), CHUNK)
            return pltpu.async_copy(
                table_hbm.at[idx_v.at[pl.ds(off, CHUNK)]], rows_v.at[slot], gsem)

        start_gather(0, 0).wait()

        @pl.loop(np.int32(0), np.int32(n_chunks - 1))
        def loop_body(c):
            slot = lax.rem(c, jnp.int32(2))
            nxt = jnp.int32(1) - slot
            cp = start_gather(c + jnp.int32(1), nxt)
            off = pl.multiple_of(base + c * jnp.int32(CHUNK), CHUNK)
            pltpu.async_copy(rows_v.at[slot], out_hbm.at[pl.ds(off, CHUNK)], ssem).wait()
            cp.wait()
        last = n_chunks - 1
        off = pl.multiple_of(base + jnp.int32(last * CHUNK), CHUNK)
        pltpu.async_copy(rows_v.at[last % 2],
                         out_hbm.at[pl.ds(off, CHUNK)], ssem).wait()

    return sc_kernel(prev, cur, table)


def kernel(prev_ids, cur_ids, embed, proj_w):
    with jax._src.config.enable_x64(False):
        b, l = prev_ids.shape
        num_buckets = embed.shape[0]
        mult = 1000003 % num_buckets
        prev = prev_ids.astype(jnp.int32).reshape(-1)
        cur = cur_ids.astype(jnp.int32).reshape(-1)
        p = _tc_build_p(embed.T, proj_w)
        out = _sc_gather(prev, cur, p, num_buckets, mult)
        return out.reshape(b, l, proj_w.shape[0])


# speedup vs baseline: 38.3833x; 1.1272x over previous
import functools

import numpy as np

import jax
import jax.numpy as jnp
from jax import lax
from jax.experimental import pallas as pl
from jax.experimental.pallas import tpu as pltpu
from jax.experimental.pallas import tpu_sc as plsc

LANES = 16
CHUNK = 128
SLOT = 256


def _tc_build_p(embed_t, proj_w, block_k=32768):
    d, nb = embed_t.shape
    m = proj_w.shape[0]

    def body(et_ref, w_ref, p_ref):
        p_ref[...] = lax.dot_general(
            et_ref[...], w_ref[...], (((0,), (1,)), ((), ())),
            preferred_element_type=jnp.float32)

    grid = (nb + block_k - 1) // block_k
    return pl.pallas_call(
        body,
        grid=(grid,),
        in_specs=[
            pl.BlockSpec((d, block_k), lambda i: (0, i)),
            pl.BlockSpec((m, d), lambda i: (0, 0)),
        ],
        out_specs=pl.BlockSpec((block_k, m), lambda i: (i, 0)),
        out_shape=jax.ShapeDtypeStruct((nb, m), jnp.float32),
    )(embed_t, proj_w)


def _sc_gather(prev, cur, table, num_buckets, mult):
    n = prev.shape[0]
    m = table.shape[1]
    info = plsc.get_sparse_core_info()
    nc, ns = info.num_cores, info.num_subcores
    nw = nc * ns
    n_per_w = n // nw
    n_chunks = n_per_w // CHUNK

    mesh = plsc.VectorSubcoreMesh(core_axis_name="c", subcore_axis_name="s")

    @functools.partial(
        pl.kernel,
        out_type=jax.ShapeDtypeStruct((n, m), jnp.float32),
        mesh=mesh,
        scratch_types=[
            pltpu.VMEM((n_per_w,), jnp.int32),
            pltpu.VMEM((n_per_w,), jnp.int32),
            pltpu.VMEM((2, SLOT, m), jnp.float32),
            pltpu.SemaphoreType.DMA,
            pltpu.SemaphoreType.DMA,
            pltpu.SemaphoreType.DMA,
        ],
        compiler_params=pltpu.CompilerParams(use_tc_tiling_on_sc=False),
    )
    def sc_kernel(prev_hbm, cur_hbm, table_hbm, out_hbm,
                  idx_v, cur_v, rows_v, gsem0, gsem1, ssem):
        wid = lax.axis_index("s") * jnp.int32(nc) + lax.axis_index("c")
        base = pl.multiple_of(wid * jnp.int32(n_per_w), n_per_w)
        pltpu.sync_copy(prev_hbm.at[pl.ds(base, n_per_w)], idx_v)
        pltpu.sync_copy(cur_hbm.at[pl.ds(base, n_per_w)], cur_v)

        two_nb = jnp.int32(2 * num_buckets)
        one_nb = jnp.int32(num_buckets)

        @pl.loop(np.int32(0), np.int32(n_per_w // LANES), unroll=4)
        def hash_body(i):
            off = pl.multiple_of(i * jnp.int32(LANES), LANES)
            h = idx_v[pl.ds(off, LANES)] * jnp.int32(mult) + cur_v[pl.ds(off, LANES)]
            h = h - jnp.where(h >= two_nb, two_nb, jnp.int32(0))
            h = h - jnp.where(h >= one_nb, one_nb, jnp.int32(0))
            idx_v[pl.ds(off, LANES)] = h

        gsems = (gsem0, gsem1)

        def start_gather(sl, slot):
            off = pl.multiple_of(sl * jnp.int32(SLOT), SLOT)
            return [
                pltpu.async_copy(
                    table_hbm.at[idx_v.at[pl.ds(off + half * CHUNK, CHUNK)]],
                    rows_v.at[slot, pl.ds(half * CHUNK, CHUNK)], gsems[half])
                for half in range(SLOT // CHUNK)
            ]

        n_slots = n_per_w // SLOT
        for cp in start_gather(0, 0):
            cp.wait()

        @pl.loop(np.int32(0), np.int32(n_slots - 1))
        def loop_body(sl):
            slot = lax.rem(sl, jnp.int32(2))
            nxt = jnp.int32(1) - slot
            cps = start_gather(sl + jnp.int32(1), nxt)
            off = pl.multiple_of(base + sl * jnp.int32(SLOT), SLOT)
            pltpu.async_copy(rows_v.at[slot], out_hbm.at[pl.ds(off, SLOT)], ssem).wait()
            for cp in cps:
                cp.wait()
        last = n_slots - 1
        off = pl.multiple_of(base + jnp.int32(last * SLOT), SLOT)
        pltpu.async_copy(rows_v.at[last % 2],
                         out_hbm.at[pl.ds(off, SLOT)], ssem).wait()

    return sc_kernel(prev, cur, table)


def kernel(prev_ids, cur_ids, embed, proj_w):
    with jax._src.config.enable_x64(False):
        b, l = prev_ids.shape
        num_buckets = embed.shape[0]
        mult = 1000003 % num_buckets
        prev = prev_ids.astype(jnp.int32).reshape(-1)
        cur = cur_ids.astype(jnp.int32).reshape(-1)
        p = _tc_build_p(embed.T, proj_w)
        out = _sc_gather(prev, cur, p, num_buckets, mult)
        return out.reshape(b, l, proj_w.shape[0])


# speedup vs baseline: 39.1432x; 1.0198x over previous
import functools

import numpy as np

import jax
import jax.numpy as jnp
from jax import lax
from jax.experimental import pallas as pl
from jax.experimental.pallas import tpu as pltpu
from jax.experimental.pallas import tpu_sc as plsc

LANES = 16
CHUNK = 128
SLOT = 256
NSLOTS = 3


def _tc_build_p(embed_t, proj_w, block_k=32768):
    d, nb = embed_t.shape
    m = proj_w.shape[0]

    def body(et_ref, w_ref, p_ref):
        p_ref[...] = lax.dot_general(
            et_ref[...], w_ref[...], (((0,), (1,)), ((), ())),
            preferred_element_type=jnp.float32)

    grid = (nb + block_k - 1) // block_k
    return pl.pallas_call(
        body,
        grid=(grid,),
        in_specs=[
            pl.BlockSpec((d, block_k), lambda i: (0, i)),
            pl.BlockSpec((m, d), lambda i: (0, 0)),
        ],
        out_specs=pl.BlockSpec((block_k, m), lambda i: (i, 0)),
        out_shape=jax.ShapeDtypeStruct((nb, m), jnp.float32),
    )(embed_t, proj_w)


def _sc_mesh_info():
    info = plsc.get_sparse_core_info()
    return info.num_cores, info.num_subcores


def _sc_hash(prev, cur, num_buckets, mult):
    n = prev.shape[0]
    nc, ns = _sc_mesh_info()
    n_per_w = n // (nc * ns)

    mesh = plsc.VectorSubcoreMesh(core_axis_name="c", subcore_axis_name="s")

    @functools.partial(
        pl.kernel,
        out_type=jax.ShapeDtypeStruct((n,), jnp.int32),
        mesh=mesh,
        scratch_types=[
            pltpu.VMEM((n_per_w,), jnp.int32),
            pltpu.VMEM((n_per_w,), jnp.int32),
        ],
        compiler_params=pltpu.CompilerParams(use_tc_tiling_on_sc=False),
    )
    def hash_kernel(prev_hbm, cur_hbm, idx_hbm, idx_v, cur_v):
        wid = lax.axis_index("s") * jnp.int32(nc) + lax.axis_index("c")
        base = pl.multiple_of(wid * jnp.int32(n_per_w), n_per_w)
        pltpu.sync_copy(prev_hbm.at[pl.ds(base, n_per_w)], idx_v)
        pltpu.sync_copy(cur_hbm.at[pl.ds(base, n_per_w)], cur_v)

        two_nb = jnp.int32(2 * num_buckets)
        one_nb = jnp.int32(num_buckets)

        @pl.loop(np.int32(0), np.int32(n_per_w // LANES), unroll=4)
        def hash_body(i):
            off = pl.multiple_of(i * jnp.int32(LANES), LANES)
            h = idx_v[pl.ds(off, LANES)] * jnp.int32(mult) + cur_v[pl.ds(off, LANES)]
            h = h - jnp.where(h >= two_nb, two_nb, jnp.int32(0))
            h = h - jnp.where(h >= one_nb, one_nb, jnp.int32(0))
            idx_v[pl.ds(off, LANES)] = h

        pltpu.sync_copy(idx_v, idx_hbm.at[pl.ds(base, n_per_w)])

    return hash_kernel(prev, cur)


def _sc_gather(idx, table):
    n = idx.shape[0]
    m = table.shape[1]
    nc, ns = _sc_mesh_info()
    n_per_w = n // (nc * ns)
    n_slots = n_per_w // SLOT

    mesh = plsc.VectorSubcoreMesh(core_axis_name="c", subcore_axis_name="s")

    @functools.partial(
        pl.kernel,
        out_type=jax.ShapeDtypeStruct((n, m), jnp.float32),
        mesh=mesh,
        scratch_types=[
            pltpu.VMEM((n_per_w,), jnp.int32),
            pltpu.VMEM((NSLOTS, SLOT, m), jnp.float32),
            pltpu.SemaphoreType.DMA,
            pltpu.SemaphoreType.DMA,
            pltpu.SemaphoreType.DMA,
        ],
        compiler_params=pltpu.CompilerParams(use_tc_tiling_on_sc=False),
    )
    def gather_kernel(idx_hbm, table_hbm, out_hbm,
                      idx_v, rows_v, gsem0, gsem1, ssem):
        wid = lax.axis_index("s") * jnp.int32(nc) + lax.axis_index("c")
        base = pl.multiple_of(wid * jnp.int32(n_per_w), n_per_w)
        pltpu.sync_copy(idx_hbm.at[pl.ds(base, n_per_w)], idx_v)

        gsems = (gsem0, gsem1)

        def start_gather(sl, slot):
            off = pl.multiple_of(sl * jnp.int32(SLOT), SLOT)
            return [
                pltpu.async_copy(
                    table_hbm.at[idx_v.at[pl.ds(off + half * CHUNK, CHUNK)]],
                    rows_v.at[slot, pl.ds(half * CHUNK, CHUNK)], gsems[half])
                for half in range(SLOT // CHUNK)
            ]

        for cp in start_gather(0, 0):
            cp.wait()

        @pl.loop(np.int32(0), np.int32(n_slots - 1))
        def loop_body(sl):
            slot = lax.rem(sl, jnp.int32(NSLOTS))
            nxt = lax.rem(sl + jnp.int32(1), jnp.int32(NSLOTS))
            cps = start_gather(sl + jnp.int32(1), nxt)
            off = pl.multiple_of(base + sl * jnp.int32(SLOT), SLOT)
            pltpu.async_copy(rows_v.at[slot], out_hbm.at[pl.ds(off, SLOT)], ssem).wait()
            for cp in cps:
                cp.wait()
        last = n_slots - 1
        off = pl.multiple_of(base + jnp.int32(last * SLOT), SLOT)
        pltpu.async_copy(rows_v.at[last % NSLOTS],
                         out_hbm.at[pl.ds(off, SLOT)], ssem).wait()

    return gather_kernel(idx, table)


def kernel(prev_ids, cur_ids, embed, proj_w):
    with jax._src.config.enable_x64(False):
        b, l = prev_ids.shape
        num_buckets = embed.shape[0]
        mult = 1000003 % num_buckets
        prev = prev_ids.astype(jnp.int32).reshape(-1)
        cur = cur_ids.astype(jnp.int32).reshape(-1)
        idx = _sc_hash(prev, cur, num_buckets, mult)
        p = _tc_build_p(embed.T, proj_w)
        out = _sc_gather(idx, p)
        return out.reshape(b, l, proj_w.shape[0])
